# baseline (speedup 1.0000x reference)
import numpy as np
import jax
import jax.numpy as jnp
from jax import lax
from jax.experimental import pallas as pl
from jax.experimental.pallas import tpu as pltpu

_PAD_TOKEN_ID = 0
_MASK_TOKEN_ID = 103
_VOCAB_SIZE = 50000
_SPAN = _VOCAB_SIZE - _MASK_TOKEN_ID
_MULT = ((65536 % _SPAN) * (65536 % _SPAN)) % _SPAN
_B16 = 65536 % _SPAN
_TH80 = 6710886
_TH10 = 838860

_ROTS = ((13, 15, 26, 6), (17, 29, 16, 24))


def _np_tf20(k0, k1, x0, x1):
    M = 0xFFFFFFFF
    ks = [int(k0), int(k1), (int(k0) ^ int(k1) ^ 0x1BD11BDA) & M]
    x0 = (int(x0) + ks[0]) & M
    x1 = (int(x1) + ks[1]) & M
    for j in range(5):
        for r in _ROTS[j % 2]:
            x0 = (x0 + x1) & M
            x1 = ((x1 << r) | (x1 >> (32 - r))) & M
            x1 ^= x0
        x0 = (x0 + ks[(j + 1) % 3]) & M
        x1 = (x1 + ks[(j + 2) % 3] + j + 1) & M
    return x0, x1


def _derive_subkeys():
    base = (0, 1234)
    kb = _np_tf20(base[0], base[1], 0, 0)
    k80 = _np_tf20(base[0], base[1], 0, 1)
    k10 = _np_tf20(base[0], base[1], 0, 2)
    kw = _np_tf20(base[0], base[1], 0, 3)
    kw1 = _np_tf20(kw[0], kw[1], 0, 0)
    kw2 = _np_tf20(kw[0], kw[1], 0, 1)
    return kb, k80, k10, kw1, kw2


_KB, _K80, _K10, _KW1, _KW2 = _derive_subkeys()


def _i32(v):
    v = int(v) & 0xFFFFFFFF
    return np.int32(v - 0x100000000 if v >= 0x80000000 else v)


def _lsr(x, d):
    return lax.shift_right_logical(x, np.int32(d))


def _tf20_bits(kpair, x1):
    k0, k1 = int(kpair[0]), int(kpair[1])
    ks = (k0, k1, (k0 ^ k1 ^ 0x1BD11BDA) & 0xFFFFFFFF)
    x0v = x1 + _i32(ks[1] + ks[0])
    x1v = x1 + _i32(ks[1])
    x1v = (x1v << np.int32(13)) | _lsr(x1v, 19)
    x1v = x1v ^ x0v
    first = True
    for j in range(5):
        rots = _ROTS[j % 2]
        for idx, r in enumerate(rots):
            if first and idx == 0:
                first = False
                continue
            x0v = x0v + x1v
            x1v = (x1v << np.int32(r)) | _lsr(x1v, 32 - r)
            x1v = x1v ^ x0v
        x0v = x0v + _i32(ks[(j + 1) % 3])
        x1v = x1v + _i32(ks[(j + 2) % 3] + j + 1)
    return x0v ^ x1v


def _mod_span(t):
    q = (t.astype(jnp.float32) * np.float32(1.0 / _SPAN)).astype(jnp.int32)
    r = t - q * np.int32(_SPAN)
    r = jnp.where(r < 0, r + np.int32(_SPAN), r)
    r = jnp.where(r >= np.int32(_SPAN), r - np.int32(_SPAN), r)
    return r


def _umod_span(x):
    xh = _lsr(x, 16)
    xl = x & np.int32(0xFFFF)
    xh = jnp.where(xh >= np.int32(_SPAN), xh - np.int32(_SPAN), xh)
    return _mod_span(xh * np.int32(_B16) + xl)


_BLOCK_ROWS = 64
_L = 2048


def _mask_kernel(inp_ref, msk_ref, sp_ref, out_inp_ref, out_msk_ref, out_lbl_ref):
    base = pl.program_id(0) * np.int32(_BLOCK_ROWS)
    rows = lax.broadcasted_iota(jnp.int32, (_BLOCK_ROWS, _L), 0)
    cols = lax.broadcasted_iota(jnp.int32, (_BLOCK_ROWS, _L), 1)
    i = (base + rows) * np.int32(_L) + cols

    inp = inp_ref[...]
    msk = msk_ref[...]
    sp = sp_ref[...]

    bits_b = _tf20_bits(_KB, i)
    masked = (bits_b >= 0) & jnp.logical_not(sp)

    bits_80 = _tf20_bits(_K80, i)
    replaced = (_lsr(bits_80, 9) <= np.int32(_TH80)) & masked

    bits_10 = _tf20_bits(_K10, i)
    is_rand = (_lsr(bits_10, 9) <= np.int32(_TH10)) & masked & jnp.logical_not(replaced)

    hi = _umod_span(_tf20_bits(_KW1, i))
    lo = _umod_span(_tf20_bits(_KW2, i))
    words = np.int32(_MASK_TOKEN_ID) + _mod_span(hi * np.int32(_MULT) + lo)

    out_msk_ref[...] = jnp.where(masked, np.int32(0), msk)
    out_lbl_ref[...] = jnp.where(masked, inp, np.int32(_PAD_TOKEN_ID))
    out = jnp.where(replaced, np.int32(_MASK_TOKEN_ID), inp)
    out_inp_ref[...] = jnp.where(is_rand, words, out)


def kernel(inputs, masks, special_tokens_mask):
    B, L = inputs.shape
    grid = (B // _BLOCK_ROWS,)
    blk = lambda: pl.BlockSpec((_BLOCK_ROWS, L), lambda b: (b, 0))
    out_shapes = (
        jax.ShapeDtypeStruct((B, L), jnp.int32),
        jax.ShapeDtypeStruct((B, L), jnp.int32),
        jax.ShapeDtypeStruct((B, L), jnp.int32),
    )
    inputs_out, masks_out, labels_out = pl.pallas_call(
        _mask_kernel,
        grid=grid,
        in_specs=[blk(), blk(), blk()],
        out_specs=(blk(), blk(), blk()),
        out_shape=out_shapes,
        compiler_params=pltpu.CompilerParams(
            dimension_semantics=("arbitrary",),
        ),
    )(inputs, masks, special_tokens_mask)
    return (inputs_out, masks_out, labels_out)


# speedup vs baseline: 1.0021x; 1.0021x over previous
import numpy as np
import jax
import jax.numpy as jnp
from jax import lax
from jax.experimental import pallas as pl
from jax.experimental.pallas import tpu as pltpu

_PAD_TOKEN_ID = 0
_MASK_TOKEN_ID = 103
_VOCAB_SIZE = 50000
_SPAN = _VOCAB_SIZE - _MASK_TOKEN_ID
_MULT = ((65536 % _SPAN) * (65536 % _SPAN)) % _SPAN
_B16 = 65536 % _SPAN
_TH80 = 6710886
_TH10 = 838860

_ROTS = ((13, 15, 26, 6), (17, 29, 16, 24))


def _np_tf20(k0, k1, x0, x1):
    M = 0xFFFFFFFF
    ks = [int(k0), int(k1), (int(k0) ^ int(k1) ^ 0x1BD11BDA) & M]
    x0 = (int(x0) + ks[0]) & M
    x1 = (int(x1) + ks[1]) & M
    for j in range(5):
        for r in _ROTS[j % 2]:
            x0 = (x0 + x1) & M
            x1 = ((x1 << r) | (x1 >> (32 - r))) & M
            x1 ^= x0
        x0 = (x0 + ks[(j + 1) % 3]) & M
        x1 = (x1 + ks[(j + 2) % 3] + j + 1) & M
    return x0, x1


def _derive_subkeys():
    base = (0, 1234)
    kb = _np_tf20(base[0], base[1], 0, 0)
    k80 = _np_tf20(base[0], base[1], 0, 1)
    k10 = _np_tf20(base[0], base[1], 0, 2)
    kw = _np_tf20(base[0], base[1], 0, 3)
    kw1 = _np_tf20(kw[0], kw[1], 0, 0)
    kw2 = _np_tf20(kw[0], kw[1], 0, 1)
    return kb, k80, k10, kw1, kw2


_KB, _K80, _K10, _KW1, _KW2 = _derive_subkeys()


def _i32(v):
    v = int(v) & 0xFFFFFFFF
    return np.int32(v - 0x100000000 if v >= 0x80000000 else v)


def _lsr(x, d):
    return lax.shift_right_logical(x, np.int32(d))


def _tf20_bits(kpair, x1):
    k0, k1 = int(kpair[0]), int(kpair[1])
    ks = (k0, k1, (k0 ^ k1 ^ 0x1BD11BDA) & 0xFFFFFFFF)
    x0v = x1 + _i32(ks[1] + ks[0])
    x1v = x1 + _i32(ks[1])
    x1v = (x1v << np.int32(13)) | _lsr(x1v, 19)
    x1v = x1v ^ x0v
    first = True
    for j in range(5):
        rots = _ROTS[j % 2]
        for idx, r in enumerate(rots):
            if first and idx == 0:
                first = False
                continue
            x0v = x0v + x1v
            x1v = (x1v << np.int32(r)) | _lsr(x1v, 32 - r)
            x1v = x1v ^ x0v
        x0v = x0v + _i32(ks[(j + 1) % 3])
        x1v = x1v + _i32(ks[(j + 2) % 3] + j + 1)
    return x0v ^ x1v


def _mod_span(t):
    q = (t.astype(jnp.float32) * np.float32(1.0 / _SPAN)).astype(jnp.int32)
    r = t - q * np.int32(_SPAN)
    r = jnp.where(r < 0, r + np.int32(_SPAN), r)
    r = jnp.where(r >= np.int32(_SPAN), r - np.int32(_SPAN), r)
    return r


def _umod_span(x):
    xh = _lsr(x, 16)
    xl = x & np.int32(0xFFFF)
    xh = jnp.where(xh >= np.int32(_SPAN), xh - np.int32(_SPAN), xh)
    return _mod_span(xh * np.int32(_B16) + xl)


_BLOCK_ROWS = 8
_L = 2048


def _mask_kernel(inp_ref, msk_ref, sp_ref, out_inp_ref, out_msk_ref, out_lbl_ref):
    base = pl.program_id(0) * np.int32(_BLOCK_ROWS)
    rows = lax.broadcasted_iota(jnp.int32, (_BLOCK_ROWS, _L), 0)
    cols = lax.broadcasted_iota(jnp.int32, (_BLOCK_ROWS, _L), 1)
    i = (base + rows) * np.int32(_L) + cols

    inp = inp_ref[...]
    msk = msk_ref[...]
    sp = sp_ref[...]

    bits_b = _tf20_bits(_KB, i)
    masked = (bits_b >= 0) & jnp.logical_not(sp)

    bits_80 = _tf20_bits(_K80, i)
    replaced = (_lsr(bits_80, 9) <= np.int32(_TH80)) & masked

    bits_10 = _tf20_bits(_K10, i)
    is_rand = (_lsr(bits_10, 9) <= np.int32(_TH10)) & masked & jnp.logical_not(replaced)

    hi = _umod_span(_tf20_bits(_KW1, i))
    lo = _umod_span(_tf20_bits(_KW2, i))
    words = np.int32(_MASK_TOKEN_ID) + _mod_span(hi * np.int32(_MULT) + lo)

    out_msk_ref[...] = jnp.where(masked, np.int32(0), msk)
    out_lbl_ref[...] = jnp.where(masked, inp, np.int32(_PAD_TOKEN_ID))
    out = jnp.where(replaced, np.int32(_MASK_TOKEN_ID), inp)
    out_inp_ref[...] = jnp.where(is_rand, words, out)


def kernel(inputs, masks, special_tokens_mask):
    B, L = inputs.shape
    grid = (B // _BLOCK_ROWS,)
    blk = lambda: pl.BlockSpec((_BLOCK_ROWS, L), lambda b: (b, 0))
    out_shapes = (
        jax.ShapeDtypeStruct((B, L), jnp.int32),
        jax.ShapeDtypeStruct((B, L), jnp.int32),
        jax.ShapeDtypeStruct((B, L), jnp.int32),
    )
    inputs_out, masks_out, labels_out = pl.pallas_call(
        _mask_kernel,
        grid=grid,
        in_specs=[blk(), blk(), blk()],
        out_specs=(blk(), blk(), blk()),
        out_shape=out_shapes,
        compiler_params=pltpu.CompilerParams(
            dimension_semantics=("arbitrary",),
        ),
    )(inputs, masks, special_tokens_mask)
    return (inputs_out, masks_out, labels_out)


# speedup vs baseline: 1.0230x; 1.0209x over previous
import numpy as np
import jax
import jax.numpy as jnp
from jax import lax
from jax.experimental import pallas as pl
from jax.experimental.pallas import tpu as pltpu

_PAD_TOKEN_ID = 0
_MASK_TOKEN_ID = 103
_VOCAB_SIZE = 50000
_SPAN = _VOCAB_SIZE - _MASK_TOKEN_ID
_MULT = ((65536 % _SPAN) * (65536 % _SPAN)) % _SPAN
_B16 = 65536 % _SPAN
_TH80 = 6710886
_TH10 = 838860

_ROTS = ((13, 15, 26, 6), (17, 29, 16, 24))


def _np_tf20(k0, k1, x0, x1):
    M = 0xFFFFFFFF
    ks = [int(k0), int(k1), (int(k0) ^ int(k1) ^ 0x1BD11BDA) & M]
    x0 = (int(x0) + ks[0]) & M
    x1 = (int(x1) + ks[1]) & M
    for j in range(5):
        for r in _ROTS[j % 2]:
            x0 = (x0 + x1) & M
            x1 = ((x1 << r) | (x1 >> (32 - r))) & M
            x1 ^= x0
        x0 = (x0 + ks[(j + 1) % 3]) & M
        x1 = (x1 + ks[(j + 2) % 3] + j + 1) & M
    return x0, x1


def _derive_subkeys():
    base = (0, 1234)
    kb = _np_tf20(base[0], base[1], 0, 0)
    k80 = _np_tf20(base[0], base[1], 0, 1)
    k10 = _np_tf20(base[0], base[1], 0, 2)
    kw = _np_tf20(base[0], base[1], 0, 3)
    kw1 = _np_tf20(kw[0], kw[1], 0, 0)
    kw2 = _np_tf20(kw[0], kw[1], 0, 1)
    return kb, k80, k10, kw1, kw2


_KB, _K80, _K10, _KW1, _KW2 = _derive_subkeys()


def _i32(v):
    v = int(v) & 0xFFFFFFFF
    return np.int32(v - 0x100000000 if v >= 0x80000000 else v)


def _lsr(x, d):
    return lax.shift_right_logical(x, np.int32(d))


def _tf20_bits(kpair, x1):
    k0, k1 = int(kpair[0]), int(kpair[1])
    ks = (k0, k1, (k0 ^ k1 ^ 0x1BD11BDA) & 0xFFFFFFFF)
    x0v = x1 + _i32(ks[1] + ks[0])
    x1v = x1 + _i32(ks[1])
    x1v = (x1v << np.int32(13)) | _lsr(x1v, 19)
    x1v = x1v ^ x0v
    first = True
    for j in range(5):
        rots = _ROTS[j % 2]
        for idx, r in enumerate(rots):
            if first and idx == 0:
                first = False
                continue
            x0v = x0v + x1v
            x1v = (x1v << np.int32(r)) | _lsr(x1v, 32 - r)
            x1v = x1v ^ x0v
        x0v = x0v + _i32(ks[(j + 1) % 3])
        x1v = x1v + _i32(ks[(j + 2) % 3] + j + 1)
    return x0v ^ x1v


def _mod_span(t):
    q = (t.astype(jnp.float32) * np.float32(1.0 / _SPAN)
         - np.float32(0.25)).astype(jnp.int32)
    r = t - q * np.int32(_SPAN)
    r = jnp.where(r >= np.int32(_SPAN), r - np.int32(_SPAN), r)
    return r


def _umod_span(x):
    xh = _lsr(x, 16)
    xl = x & np.int32(0xFFFF)
    return _mod_span(xh * np.int32(_B16) + xl)


_BLOCK_ROWS = 8
_L = 2048


def _mask_kernel(inp_ref, msk_ref, sp_ref, out_inp_ref, out_msk_ref, out_lbl_ref):
    base = pl.program_id(0) * np.int32(_BLOCK_ROWS)
    rows = lax.broadcasted_iota(jnp.int32, (_BLOCK_ROWS, _L), 0)
    cols = lax.broadcasted_iota(jnp.int32, (_BLOCK_ROWS, _L), 1)
    i = (base + rows) * np.int32(_L) + cols

    inp = inp_ref[...]
    msk = msk_ref[...]
    sp = sp_ref[...]

    bits_b = _tf20_bits(_KB, i)
    masked = (bits_b >= 0) & jnp.logical_not(sp)

    bits_80 = _tf20_bits(_K80, i)
    replaced = (_lsr(bits_80, 9) <= np.int32(_TH80)) & masked

    bits_10 = _tf20_bits(_K10, i)
    is_rand = (_lsr(bits_10, 9) <= np.int32(_TH10)) & masked & jnp.logical_not(replaced)

    hi = _umod_span(_tf20_bits(_KW1, i))
    lo = _umod_span(_tf20_bits(_KW2, i))
    words = np.int32(_MASK_TOKEN_ID) + _mod_span(hi * np.int32(_MULT) + lo)

    out_msk_ref[...] = jnp.where(masked, np.int32(0), msk)
    out_lbl_ref[...] = jnp.where(masked, inp, np.int32(_PAD_TOKEN_ID))
    out = jnp.where(replaced, np.int32(_MASK_TOKEN_ID), inp)
    out_inp_ref[...] = jnp.where(is_rand, words, out)


def kernel(inputs, masks, special_tokens_mask):
    B, L = inputs.shape
    grid = (B // _BLOCK_ROWS,)
    blk = lambda: pl.BlockSpec((_BLOCK_ROWS, L), lambda b: (b, 0))
    out_shapes = (
        jax.ShapeDtypeStruct((B, L), jnp.int32),
        jax.ShapeDtypeStruct((B, L), jnp.int32),
        jax.ShapeDtypeStruct((B, L), jnp.int32),
    )
    inputs_out, masks_out, labels_out = pl.pallas_call(
        _mask_kernel,
        grid=grid,
        in_specs=[blk(), blk(), blk()],
        out_specs=(blk(), blk(), blk()),
        out_shape=out_shapes,
        compiler_params=pltpu.CompilerParams(
            dimension_semantics=("arbitrary",),
        ),
    )(inputs, masks, special_tokens_mask)
    return (inputs_out, masks_out, labels_out)


# speedup vs baseline: 1.0280x; 1.0048x over previous
import numpy as np
import jax
import jax.numpy as jnp
from jax import lax
from jax.experimental import pallas as pl
from jax.experimental.pallas import tpu as pltpu

_PAD_TOKEN_ID = 0
_MASK_TOKEN_ID = 103
_VOCAB_SIZE = 50000
_SPAN = _VOCAB_SIZE - _MASK_TOKEN_ID
_MULT = ((65536 % _SPAN) * (65536 % _SPAN)) % _SPAN
_B16 = 65536 % _SPAN
_TH80 = 6710886
_TH10 = 838860

_ROTS = ((13, 15, 26, 6), (17, 29, 16, 24))


def _np_tf20(k0, k1, x0, x1):
    M = 0xFFFFFFFF
    ks = [int(k0), int(k1), (int(k0) ^ int(k1) ^ 0x1BD11BDA) & M]
    x0 = (int(x0) + ks[0]) & M
    x1 = (int(x1) + ks[1]) & M
    for j in range(5):
        for r in _ROTS[j % 2]:
            x0 = (x0 + x1) & M
            x1 = ((x1 << r) | (x1 >> (32 - r))) & M
            x1 ^= x0
        x0 = (x0 + ks[(j + 1) % 3]) & M
        x1 = (x1 + ks[(j + 2) % 3] + j + 1) & M
    return x0, x1


def _derive_subkeys():
    base = (0, 1234)
    kb = _np_tf20(base[0], base[1], 0, 0)
    k80 = _np_tf20(base[0], base[1], 0, 1)
    k10 = _np_tf20(base[0], base[1], 0, 2)
    kw = _np_tf20(base[0], base[1], 0, 3)
    kw1 = _np_tf20(kw[0], kw[1], 0, 0)
    kw2 = _np_tf20(kw[0], kw[1], 0, 1)
    return kb, k80, k10, kw1, kw2


_KB, _K80, _K10, _KW1, _KW2 = _derive_subkeys()


def _i32(v):
    v = int(v) & 0xFFFFFFFF
    return np.int32(v - 0x100000000 if v >= 0x80000000 else v)


def _lsr(x, d):
    return lax.shift_right_logical(x, np.int32(d))


def _tf20_bits(kpair, x1):
    k0, k1 = int(kpair[0]), int(kpair[1])
    ks = (k0, k1, (k0 ^ k1 ^ 0x1BD11BDA) & 0xFFFFFFFF)
    x0v = x1 + _i32(ks[1] + ks[0])
    x1v = x1 + _i32(ks[1])
    x1v = (x1v << np.int32(13)) | _lsr(x1v, 19)
    x1v = x1v ^ x0v
    first = True
    for j in range(5):
        rots = _ROTS[j % 2]
        for idx, r in enumerate(rots):
            if first and idx == 0:
                first = False
                continue
            x0v = x0v + x1v
            x1v = (x1v << np.int32(r)) | _lsr(x1v, 32 - r)
            x1v = x1v ^ x0v
        x0v = x0v + _i32(ks[(j + 1) % 3])
        x1v = x1v + _i32(ks[(j + 2) % 3] + j + 1)
    return x0v ^ x1v


def _mod_span(t):
    q = (t.astype(jnp.float32) * np.float32(1.0 / _SPAN)
         - np.float32(0.25)).astype(jnp.int32)
    r = t - q * np.int32(_SPAN)
    r = jnp.where(r >= np.int32(_SPAN), r - np.int32(_SPAN), r)
    return r


def _umod_span(x):
    xh = _lsr(x, 16)
    xl = x & np.int32(0xFFFF)
    return _mod_span(xh * np.int32(_B16) + xl)


_BLOCK_ROWS = 16
_L = 2048


def _mask_kernel(inp_ref, msk_ref, sp_ref, out_inp_ref, out_msk_ref, out_lbl_ref):
    base = pl.program_id(0) * np.int32(_BLOCK_ROWS)
    rows = lax.broadcasted_iota(jnp.int32, (_BLOCK_ROWS, _L), 0)
    cols = lax.broadcasted_iota(jnp.int32, (_BLOCK_ROWS, _L), 1)
    i = (base + rows) * np.int32(_L) + cols

    inp = inp_ref[...]
    msk = msk_ref[...]
    sp = sp_ref[...]

    bits_b = _tf20_bits(_KB, i)
    masked = (bits_b >= 0) & jnp.logical_not(sp)

    bits_80 = _tf20_bits(_K80, i)
    replaced = (_lsr(bits_80, 9) <= np.int32(_TH80)) & masked

    bits_10 = _tf20_bits(_K10, i)
    is_rand = (_lsr(bits_10, 9) <= np.int32(_TH10)) & masked & jnp.logical_not(replaced)

    hi = _umod_span(_tf20_bits(_KW1, i))
    lo = _umod_span(_tf20_bits(_KW2, i))
    words = np.int32(_MASK_TOKEN_ID) + _mod_span(hi * np.int32(_MULT) + lo)

    out_msk_ref[...] = jnp.where(masked, np.int32(0), msk)
    out_lbl_ref[...] = jnp.where(masked, inp, np.int32(_PAD_TOKEN_ID))
    out = jnp.where(replaced, np.int32(_MASK_TOKEN_ID), inp)
    out_inp_ref[...] = jnp.where(is_rand, words, out)


def kernel(inputs, masks, special_tokens_mask):
    B, L = inputs.shape
    grid = (B // _BLOCK_ROWS,)
    blk = lambda: pl.BlockSpec((_BLOCK_ROWS, L), lambda b: (b, 0))
    out_shapes = (
        jax.ShapeDtypeStruct((B, L), jnp.int32),
        jax.ShapeDtypeStruct((B, L), jnp.int32),
        jax.ShapeDtypeStruct((B, L), jnp.int32),
    )
    inputs_out, masks_out, labels_out = pl.pallas_call(
        _mask_kernel,
        grid=grid,
        in_specs=[blk(), blk(), blk()],
        out_specs=(blk(), blk(), blk()),
        out_shape=out_shapes,
        compiler_params=pltpu.CompilerParams(
            dimension_semantics=("arbitrary",),
        ),
    )(inputs, masks, special_tokens_mask)
    return (inputs_out, masks_out, labels_out)


# speedup vs baseline: 1.0290x; 1.0010x over previous
import numpy as np
import jax
import jax.numpy as jnp
from jax import lax
from jax.experimental import pallas as pl
from jax.experimental.pallas import tpu as pltpu

_PAD_TOKEN_ID = 0
_MASK_TOKEN_ID = 103
_VOCAB_SIZE = 50000
_SPAN = _VOCAB_SIZE - _MASK_TOKEN_ID
_MULT = ((65536 % _SPAN) * (65536 % _SPAN)) % _SPAN
_B16 = 65536 % _SPAN
_TH80 = 6710886
_TH10 = 838860

_ROTS = ((13, 15, 26, 6), (17, 29, 16, 24))


def _np_tf20(k0, k1, x0, x1):
    M = 0xFFFFFFFF
    ks = [int(k0), int(k1), (int(k0) ^ int(k1) ^ 0x1BD11BDA) & M]
    x0 = (int(x0) + ks[0]) & M
    x1 = (int(x1) + ks[1]) & M
    for j in range(5):
        for r in _ROTS[j % 2]:
            x0 = (x0 + x1) & M
            x1 = ((x1 << r) | (x1 >> (32 - r))) & M
            x1 ^= x0
        x0 = (x0 + ks[(j + 1) % 3]) & M
        x1 = (x1 + ks[(j + 2) % 3] + j + 1) & M
    return x0, x1


def _derive_subkeys():
    base = (0, 1234)
    kb = _np_tf20(base[0], base[1], 0, 0)
    k80 = _np_tf20(base[0], base[1], 0, 1)
    k10 = _np_tf20(base[0], base[1], 0, 2)
    kw = _np_tf20(base[0], base[1], 0, 3)
    kw1 = _np_tf20(kw[0], kw[1], 0, 0)
    kw2 = _np_tf20(kw[0], kw[1], 0, 1)
    return kb, k80, k10, kw1, kw2


_KB, _K80, _K10, _KW1, _KW2 = _derive_subkeys()


def _i32(v):
    v = int(v) & 0xFFFFFFFF
    return np.int32(v - 0x100000000 if v >= 0x80000000 else v)


def _lsr(x, d):
    return lax.shift_right_logical(x, np.int32(d))


def _tf20_bits(kpair, x1):
    k0, k1 = int(kpair[0]), int(kpair[1])
    ks = (k0, k1, (k0 ^ k1 ^ 0x1BD11BDA) & 0xFFFFFFFF)
    x0v = x1 + _i32(ks[1] + ks[0])
    x1v = x1 + _i32(ks[1])
    x1v = (x1v << np.int32(13)) | _lsr(x1v, 19)
    x1v = x1v ^ x0v
    first = True
    for j in range(5):
        rots = _ROTS[j % 2]
        for idx, r in enumerate(rots):
            if first and idx == 0:
                first = False
                continue
            x0v = x0v + x1v
            x1v = (x1v << np.int32(r)) | _lsr(x1v, 32 - r)
            x1v = x1v ^ x0v
        x0v = x0v + _i32(ks[(j + 1) % 3])
        x1v = x1v + _i32(ks[(j + 2) % 3] + j + 1)
    return x0v ^ x1v


def _mod_span(t):
    q = (t.astype(jnp.float32) * np.float32(1.0 / _SPAN)
         - np.float32(0.25)).astype(jnp.int32)
    r = t - q * np.int32(_SPAN)
    r = jnp.where(r >= np.int32(_SPAN), r - np.int32(_SPAN), r)
    return r


def _umod_span(x):
    xh = _lsr(x, 16)
    xl = x & np.int32(0xFFFF)
    return _mod_span(xh * np.int32(_B16) + xl)


_BLOCK_ROWS = 16
_L = 2048


def _mask_kernel(inp_ref, msk_ref, sp_ref, out_inp_ref, out_msk_ref, out_lbl_ref):
    base = pl.program_id(0) * np.int32(_BLOCK_ROWS)
    rows = lax.broadcasted_iota(jnp.int32, (_BLOCK_ROWS, _L), 0)
    cols = lax.broadcasted_iota(jnp.int32, (_BLOCK_ROWS, _L), 1)
    i = (base + rows) * np.int32(_L) + cols

    inp = inp_ref[...]
    msk = msk_ref[...]
    sp = sp_ref[...]

    bits_b = _tf20_bits(_KB, i)
    masked = (bits_b >= 0) & jnp.logical_not(sp)

    bits_80 = _tf20_bits(_K80, i)
    replaced = (_lsr(bits_80, 9) <= np.int32(_TH80)) & masked

    bits_10 = _tf20_bits(_K10, i)
    is_rand = (_lsr(bits_10, 9) <= np.int32(_TH10)) & masked & jnp.logical_not(replaced)

    hi = _umod_span(_tf20_bits(_KW1, i))
    lo = _umod_span(_tf20_bits(_KW2, i))
    words = np.int32(_MASK_TOKEN_ID) + _mod_span(hi * np.int32(_MULT) + lo)

    out_msk_ref[...] = jnp.where(masked, np.int32(0), msk)
    out_lbl_ref[...] = jnp.where(masked, inp, np.int32(_PAD_TOKEN_ID))
    out = jnp.where(replaced, np.int32(_MASK_TOKEN_ID), inp)
    out_inp_ref[...] = jnp.where(is_rand, words, out)


def kernel(inputs, masks, special_tokens_mask):
    B, L = inputs.shape
    grid = (B // _BLOCK_ROWS,)
    blk = lambda: pl.BlockSpec((_BLOCK_ROWS, L), lambda b: (b, 0))
    out_shapes = (
        jax.ShapeDtypeStruct((B, L), jnp.int32),
        jax.ShapeDtypeStruct((B, L), jnp.int32),
        jax.ShapeDtypeStruct((B, L), jnp.int32),
    )
    inputs_out, masks_out, labels_out = pl.pallas_call(
        _mask_kernel,
        grid=grid,
        in_specs=[blk(), blk(), blk()],
        out_specs=(blk(), blk(), blk()),
        out_shape=out_shapes,
        compiler_params=pltpu.CompilerParams(
            dimension_semantics=("parallel",),
        ),
    )(inputs, masks, special_tokens_mask)
    return (inputs_out, masks_out, labels_out)
